# baseline (speedup 1.0000x reference)
import functools

import jax
import jax.numpy as jnp
from jax import lax
from jax.experimental import pallas as pl
from jax.experimental.pallas import tpu as pltpu
from jax.experimental.pallas import tpu_sc as plsc

N_NODES = 10000
D = 128
N_INCID = 320000

NR = 10240
DUMMY = N_NODES
NW = 32
K = 128
C0 = 142
C1 = 16
TCH = 16 * (C0 + C1)
NP = TCH * K
RPT = NR // 16

_mesh = plsc.VectorSubcoreMesh(
    core_axis_name="c", subcore_axis_name="s", num_cores=2, num_subcores=16)


def _seg_body(with_counts, *refs):
    if with_counts:
        (x_hbm, hh, zf, zc, ones_hbm,
         part, c0part, c1part,
         idx, r0, r1, ones_v, acc, c0_sh, c1_sh,
         g0s, g1s, s0s, s1s,
         a0, a1, b0_, b1_) = refs
        sem_c0 = (a0, a1)
        sem_c1 = (b0_, b1_)
    else:
        (x_hbm, hh, zf,
         part,
         idx, r0, r1, acc,
         g0s, g1s, s0s, s1s) = refs
    rows = (r0, r1)
    sem_g = (g0s, g1s)
    sem_s = (s0s, s1s)

    cid = lax.axis_index("c")
    sid = lax.axis_index("s")
    row0 = sid * RPT
    start = jnp.where(cid == 0, sid * C0, 16 * C0 + sid * C1)
    cnt = jnp.where(cid == 0, C0, C1)

    pltpu.sync_copy(zf, acc.at[pl.ds(row0, RPT)])
    if with_counts:
        pltpu.sync_copy(zc, c0_sh.at[pl.ds(row0, RPT)])
        pltpu.sync_copy(zc, c1_sh.at[pl.ds(row0, RPT)])
        pltpu.sync_copy(ones_hbm, ones_v)
    plsc.subcore_barrier()

    NB = 2

    def issue(g, b):
        pltpu.sync_copy(hh.at[g], idx.at[b])
        pltpu.async_copy(x_hbm.at[idx.at[b, 0]], rows[b], sem_g[b])

    def drain(b):
        pltpu.make_async_copy(rows[b], acc.at[idx.at[b, 1]],
                              sem_s[b]).wait()
        if with_counts:
            pltpu.make_async_copy(ones_v, c0_sh.at[idx.at[b, 0]],
                                  sem_c0[b]).wait()
            pltpu.make_async_copy(ones_v, c1_sh.at[idx.at[b, 1]],
                                  sem_c1[b]).wait()

    def consume(g, b):
        pltpu.make_async_copy(x_hbm.at[idx.at[b, 0]], rows[b],
                              sem_g[b]).wait()
        pltpu.async_copy(rows[b], acc.at[idx.at[b, 1]], sem_s[b],
                         add=True)
        if with_counts:
            pltpu.async_copy(ones_v, c0_sh.at[idx.at[b, 0]], sem_c0[b],
                             add=True)
            pltpu.async_copy(ones_v, c1_sh.at[idx.at[b, 1]], sem_c1[b],
                             add=True)

    issue(start, 0)
    issue(start + 1, 1)

    def duo(i, carry):
        for b in range(NB):
            g = start + NB * i + b
            consume(g, b)
            gp = g + NB

            @pl.when(gp < start + cnt)
            def _():
                drain(b)
                issue(gp, b)
        return carry

    lax.fori_loop(0, cnt // NB, duo, 0)
    for b in range(NB):
        drain(b)
    plsc.subcore_barrier()

    pltpu.sync_copy(acc.at[pl.ds(row0, RPT)], part.at[cid, pl.ds(row0, RPT)])
    if with_counts:
        pltpu.sync_copy(c0_sh.at[pl.ds(row0, RPT)],
                        c0part.at[cid, pl.ds(row0, RPT)])
        pltpu.sync_copy(c1_sh.at[pl.ds(row0, RPT)],
                        c1part.at[cid, pl.ds(row0, RPT)])


_seg_counts = functools.partial(
    pl.kernel,
    mesh=_mesh,
    out_type=(
        jax.ShapeDtypeStruct((2, NR, D), jnp.float32),
        jax.ShapeDtypeStruct((2, NR), jnp.float32),
        jax.ShapeDtypeStruct((2, NR), jnp.float32),
    ),
    scratch_types=(
        [pltpu.VMEM((2, 2, K), jnp.int32)]
        + [pltpu.VMEM((K, D), jnp.float32)] * 2
        + [pltpu.VMEM((K,), jnp.float32)]
        + [pltpu.VMEM_SHARED((NR, D), jnp.float32),
           pltpu.VMEM_SHARED((NR,), jnp.float32),
           pltpu.VMEM_SHARED((NR,), jnp.float32)]
        + [pltpu.SemaphoreType.DMA] * 8
    ),
)(functools.partial(_seg_body, True))

_seg_nc = functools.partial(
    pl.kernel,
    mesh=_mesh,
    out_type=jax.ShapeDtypeStruct((2, NR, D), jnp.float32),
    scratch_types=(
        [pltpu.VMEM((2, 2, K), jnp.int32)]
        + [pltpu.VMEM((K, D), jnp.float32)] * 2
        + [pltpu.VMEM_SHARED((NR, D), jnp.float32)]
        + [pltpu.SemaphoreType.DMA] * 4
    ),
)(functools.partial(_seg_body, False))


_BR = 1024
_GRID = NR // _BR


def _mm_body(x_ref, w_ref, o_ref):
    o_ref[...] = lax.dot_general(
        x_ref[...], w_ref[...], (((1,), (1,)), ((), ())),
        preferred_element_type=jnp.float32)


_mm = pl.pallas_call(
    _mm_body,
    grid=(_GRID,),
    in_specs=[
        pl.BlockSpec((_BR, D), lambda i: (i, 0)),
        pl.BlockSpec((D, D), lambda i: (0, 0)),
    ],
    out_specs=pl.BlockSpec((_BR, D), lambda i: (i, 0)),
    out_shape=jax.ShapeDtypeStruct((NR, D), jnp.float32),
)


def _inv(c):
    return jnp.where(c > 0.0, 1.0 / jnp.maximum(c, 1e-30), 0.0)


def _mid_body(e_ref, c_ref, o_ref):
    e = e_ref[...]
    c = c_ref[...]
    o_ref[...] = (e[0] + e[1]) * _inv(c[0] + c[1])[:, None]


_mid = pl.pallas_call(
    _mid_body,
    grid=(_GRID,),
    in_specs=[
        pl.BlockSpec((2, _BR, D), lambda i: (0, i, 0)),
        pl.BlockSpec((2, _BR), lambda i: (0, i)),
    ],
    out_specs=pl.BlockSpec((_BR, D), lambda i: (i, 0)),
    out_shape=jax.ShapeDtypeStruct((NR, D), jnp.float32),
)


def _post0_body(s_ref, c_ref, b_ref, x0_ref, w1_ref, X1_ref, res_ref, x1_ref):
    s = s_ref[...]
    c = c_ref[...]
    h = (s[0] + s[1]) * _inv(c[0] + c[1])[:, None] + b_ref[...]
    h = jnp.where(h > 0.0, h, 0.01 * h)
    X1 = h + x0_ref[...]
    X1_ref[...] = X1
    res_ref[...] = x0_ref[...] + 0.5 * X1
    x1_ref[...] = lax.dot_general(
        X1, w1_ref[...], (((1,), (1,)), ((), ())),
        preferred_element_type=jnp.float32)


_post0 = pl.pallas_call(
    _post0_body,
    grid=(_GRID,),
    in_specs=[
        pl.BlockSpec((2, _BR, D), lambda i: (0, i, 0)),
        pl.BlockSpec((2, _BR), lambda i: (0, i)),
        pl.BlockSpec((1, D), lambda i: (0, 0)),
        pl.BlockSpec((_BR, D), lambda i: (i, 0)),
        pl.BlockSpec((D, D), lambda i: (0, 0)),
    ],
    out_specs=[
        pl.BlockSpec((_BR, D), lambda i: (i, 0)),
        pl.BlockSpec((_BR, D), lambda i: (i, 0)),
        pl.BlockSpec((_BR, D), lambda i: (i, 0)),
    ],
    out_shape=[
        jax.ShapeDtypeStruct((NR, D), jnp.float32),
        jax.ShapeDtypeStruct((NR, D), jnp.float32),
        jax.ShapeDtypeStruct((NR, D), jnp.float32),
    ],
)


def _post1_body(s_ref, c_ref, b_ref, x1_ref, resin_ref, o_ref):
    s = s_ref[...]
    c = c_ref[...]
    h = (s[0] + s[1]) * _inv(c[0] + c[1])[:, None] + b_ref[...]
    h = jnp.where(h > 0.0, h, 0.01 * h)
    X2 = h + x1_ref[...]
    o_ref[...] = resin_ref[...] + X2 * (1.0 / 3.0)


_post1 = pl.pallas_call(
    _post1_body,
    grid=(_GRID,),
    in_specs=[
        pl.BlockSpec((2, _BR, D), lambda i: (0, i, 0)),
        pl.BlockSpec((2, _BR), lambda i: (0, i)),
        pl.BlockSpec((1, D), lambda i: (0, 0)),
        pl.BlockSpec((_BR, D), lambda i: (i, 0)),
        pl.BlockSpec((_BR, D), lambda i: (i, 0)),
    ],
    out_specs=pl.BlockSpec((_BR, D), lambda i: (i, 0)),
    out_shape=jax.ShapeDtypeStruct((NR, D), jnp.float32),
)


def kernel(X, adj_indices, W0, b0, W1, b1):
    Xp = jnp.zeros((NR, D), jnp.float32).at[:N_NODES].set(X)
    h0 = adj_indices[0].astype(jnp.int32)
    h1 = adj_indices[1].astype(jnp.int32)
    pad = jnp.full((NP - N_INCID,), DUMMY, jnp.int32)
    h0p = jnp.concatenate([h0, pad]).reshape(TCH, K)
    h1p = jnp.concatenate([h1, pad]).reshape(TCH, K)
    h01 = jnp.stack([h0p, h1p], axis=1)
    h10 = jnp.stack([h1p, h0p], axis=1)
    zf = jnp.zeros((RPT, D), jnp.float32)
    zc = jnp.zeros((RPT,), jnp.float32)
    ones = jnp.ones((K,), jnp.float32)
    b0r = b0.reshape(1, D)
    b1r = b1.reshape(1, D)

    x0 = _mm(Xp, W0)
    E, c0p, c1p = _seg_counts(x0, h01, zf, zc, ones)
    oute = _mid(E, c1p)
    S = _seg_nc(oute, h10, zf)
    X1, res1, x1 = _post0(S, c0p, b0r, Xp, W1)
    E1 = _seg_nc(x1, h01, zf)
    oute1 = _mid(E1, c1p)
    S1 = _seg_nc(oute1, h10, zf)
    res = _post1(S1, c0p, b1r, X1, res1)
    return res[:N_NODES]


# speedup vs baseline: 1.0547x; 1.0547x over previous
import functools

import jax
import jax.numpy as jnp
from jax import lax
from jax.experimental import pallas as pl
from jax.experimental.pallas import tpu as pltpu
from jax.experimental.pallas import tpu_sc as plsc

N_NODES = 10000
D = 128
N_INCID = 320000

NR = 10240
DUMMY = N_NODES
NW = 32
K = 128
C0 = 132
C1 = 26
TCH = 16 * (C0 + C1)
NP = TCH * K
RPT = NR // 16

_mesh = plsc.VectorSubcoreMesh(
    core_axis_name="c", subcore_axis_name="s", num_cores=2, num_subcores=16)


def _seg_body(with_counts, *refs):
    if with_counts:
        (x_hbm, hh, zf, zc, ones_hbm,
         part, c0part, c1part,
         idx, r0, r1, ones_v, acc, c0_sh, c1_sh,
         g0s, g1s, s0s, s1s,
         a0, a1, b0_, b1_) = refs
        sem_c0 = (a0, a1)
        sem_c1 = (b0_, b1_)
    else:
        (x_hbm, hh, zf,
         part,
         idx, r0, r1, acc,
         g0s, g1s, s0s, s1s) = refs
    rows = (r0, r1)
    sem_g = (g0s, g1s)
    sem_s = (s0s, s1s)

    cid = lax.axis_index("c")
    sid = lax.axis_index("s")
    row0 = sid * RPT
    start = jnp.where(cid == 0, sid * C0, 16 * C0 + sid * C1)
    cnt = jnp.where(cid == 0, C0, C1)

    pltpu.sync_copy(zf, acc.at[pl.ds(row0, RPT)])
    if with_counts:
        pltpu.sync_copy(zc, c0_sh.at[pl.ds(row0, RPT)])
        pltpu.sync_copy(zc, c1_sh.at[pl.ds(row0, RPT)])
        pltpu.sync_copy(ones_hbm, ones_v)
    plsc.subcore_barrier()

    NB = 2

    def issue(g, b):
        pltpu.sync_copy(hh.at[g], idx.at[b])
        pltpu.async_copy(x_hbm.at[idx.at[b, 0]], rows[b], sem_g[b])

    def drain(b):
        pltpu.make_async_copy(rows[b], acc.at[idx.at[b, 1]],
                              sem_s[b]).wait()
        if with_counts:
            pltpu.make_async_copy(ones_v, c0_sh.at[idx.at[b, 0]],
                                  sem_c0[b]).wait()
            pltpu.make_async_copy(ones_v, c1_sh.at[idx.at[b, 1]],
                                  sem_c1[b]).wait()

    def consume(g, b):
        pltpu.make_async_copy(x_hbm.at[idx.at[b, 0]], rows[b],
                              sem_g[b]).wait()
        pltpu.async_copy(rows[b], acc.at[idx.at[b, 1]], sem_s[b],
                         add=True)
        if with_counts:
            pltpu.async_copy(ones_v, c0_sh.at[idx.at[b, 0]], sem_c0[b],
                             add=True)
            pltpu.async_copy(ones_v, c1_sh.at[idx.at[b, 1]], sem_c1[b],
                             add=True)

    issue(start, 0)
    issue(start + 1, 1)

    def duo(i, carry):
        for b in range(NB):
            g = start + NB * i + b
            consume(g, b)
            gp = g + NB

            @pl.when(gp < start + cnt)
            def _():
                drain(b)
                issue(gp, b)
        return carry

    lax.fori_loop(0, cnt // NB, duo, 0)
    for b in range(NB):
        drain(b)
    plsc.subcore_barrier()

    pltpu.sync_copy(acc.at[pl.ds(row0, RPT)], part.at[cid, pl.ds(row0, RPT)])
    if with_counts:
        pltpu.sync_copy(c0_sh.at[pl.ds(row0, RPT)],
                        c0part.at[cid, pl.ds(row0, RPT)])
        pltpu.sync_copy(c1_sh.at[pl.ds(row0, RPT)],
                        c1part.at[cid, pl.ds(row0, RPT)])


_seg_counts = functools.partial(
    pl.kernel,
    mesh=_mesh,
    out_type=(
        jax.ShapeDtypeStruct((2, NR, D), jnp.float32),
        jax.ShapeDtypeStruct((2, NR), jnp.float32),
        jax.ShapeDtypeStruct((2, NR), jnp.float32),
    ),
    scratch_types=(
        [pltpu.VMEM((2, 2, K), jnp.int32)]
        + [pltpu.VMEM((K, D), jnp.float32)] * 2
        + [pltpu.VMEM((K,), jnp.float32)]
        + [pltpu.VMEM_SHARED((NR, D), jnp.float32),
           pltpu.VMEM_SHARED((NR,), jnp.float32),
           pltpu.VMEM_SHARED((NR,), jnp.float32)]
        + [pltpu.SemaphoreType.DMA] * 8
    ),
)(functools.partial(_seg_body, True))

_seg_nc = functools.partial(
    pl.kernel,
    mesh=_mesh,
    out_type=jax.ShapeDtypeStruct((2, NR, D), jnp.float32),
    scratch_types=(
        [pltpu.VMEM((2, 2, K), jnp.int32)]
        + [pltpu.VMEM((K, D), jnp.float32)] * 2
        + [pltpu.VMEM_SHARED((NR, D), jnp.float32)]
        + [pltpu.SemaphoreType.DMA] * 4
    ),
)(functools.partial(_seg_body, False))


_BR = 1024
_GRID = NR // _BR


def _mm_body(x_ref, w_ref, o_ref):
    o_ref[...] = lax.dot_general(
        x_ref[...], w_ref[...], (((1,), (1,)), ((), ())),
        preferred_element_type=jnp.float32)


_mm = pl.pallas_call(
    _mm_body,
    grid=(_GRID,),
    in_specs=[
        pl.BlockSpec((_BR, D), lambda i: (i, 0)),
        pl.BlockSpec((D, D), lambda i: (0, 0)),
    ],
    out_specs=pl.BlockSpec((_BR, D), lambda i: (i, 0)),
    out_shape=jax.ShapeDtypeStruct((NR, D), jnp.float32),
)


def _inv(c):
    return jnp.where(c > 0.0, 1.0 / jnp.maximum(c, 1e-30), 0.0)


def _mid_body(e_ref, c_ref, o_ref):
    e = e_ref[...]
    c = c_ref[...]
    o_ref[...] = (e[0] + e[1]) * _inv(c[0] + c[1])[:, None]


_mid = pl.pallas_call(
    _mid_body,
    grid=(_GRID,),
    in_specs=[
        pl.BlockSpec((2, _BR, D), lambda i: (0, i, 0)),
        pl.BlockSpec((2, _BR), lambda i: (0, i)),
    ],
    out_specs=pl.BlockSpec((_BR, D), lambda i: (i, 0)),
    out_shape=jax.ShapeDtypeStruct((NR, D), jnp.float32),
)


def _post0_body(s_ref, c_ref, b_ref, x0_ref, w1_ref, X1_ref, res_ref, x1_ref):
    s = s_ref[...]
    c = c_ref[...]
    h = (s[0] + s[1]) * _inv(c[0] + c[1])[:, None] + b_ref[...]
    h = jnp.where(h > 0.0, h, 0.01 * h)
    X1 = h + x0_ref[...]
    X1_ref[...] = X1
    res_ref[...] = x0_ref[...] + 0.5 * X1
    x1_ref[...] = lax.dot_general(
        X1, w1_ref[...], (((1,), (1,)), ((), ())),
        preferred_element_type=jnp.float32)


_post0 = pl.pallas_call(
    _post0_body,
    grid=(_GRID,),
    in_specs=[
        pl.BlockSpec((2, _BR, D), lambda i: (0, i, 0)),
        pl.BlockSpec((2, _BR), lambda i: (0, i)),
        pl.BlockSpec((1, D), lambda i: (0, 0)),
        pl.BlockSpec((_BR, D), lambda i: (i, 0)),
        pl.BlockSpec((D, D), lambda i: (0, 0)),
    ],
    out_specs=[
        pl.BlockSpec((_BR, D), lambda i: (i, 0)),
        pl.BlockSpec((_BR, D), lambda i: (i, 0)),
        pl.BlockSpec((_BR, D), lambda i: (i, 0)),
    ],
    out_shape=[
        jax.ShapeDtypeStruct((NR, D), jnp.float32),
        jax.ShapeDtypeStruct((NR, D), jnp.float32),
        jax.ShapeDtypeStruct((NR, D), jnp.float32),
    ],
)


def _post1_body(s_ref, c_ref, b_ref, x1_ref, resin_ref, o_ref):
    s = s_ref[...]
    c = c_ref[...]
    h = (s[0] + s[1]) * _inv(c[0] + c[1])[:, None] + b_ref[...]
    h = jnp.where(h > 0.0, h, 0.01 * h)
    X2 = h + x1_ref[...]
    o_ref[...] = resin_ref[...] + X2 * (1.0 / 3.0)


_post1 = pl.pallas_call(
    _post1_body,
    grid=(_GRID,),
    in_specs=[
        pl.BlockSpec((2, _BR, D), lambda i: (0, i, 0)),
        pl.BlockSpec((2, _BR), lambda i: (0, i)),
        pl.BlockSpec((1, D), lambda i: (0, 0)),
        pl.BlockSpec((_BR, D), lambda i: (i, 0)),
        pl.BlockSpec((_BR, D), lambda i: (i, 0)),
    ],
    out_specs=pl.BlockSpec((_BR, D), lambda i: (i, 0)),
    out_shape=jax.ShapeDtypeStruct((NR, D), jnp.float32),
)


def kernel(X, adj_indices, W0, b0, W1, b1):
    Xp = jnp.zeros((NR, D), jnp.float32).at[:N_NODES].set(X)
    h0 = adj_indices[0].astype(jnp.int32)
    h1 = adj_indices[1].astype(jnp.int32)
    pad = jnp.full((NP - N_INCID,), DUMMY, jnp.int32)
    h0p = jnp.concatenate([h0, pad]).reshape(TCH, K)
    h1p = jnp.concatenate([h1, pad]).reshape(TCH, K)
    h01 = jnp.stack([h0p, h1p], axis=1)
    h10 = jnp.stack([h1p, h0p], axis=1)
    zf = jnp.zeros((RPT, D), jnp.float32)
    zc = jnp.zeros((RPT,), jnp.float32)
    ones = jnp.ones((K,), jnp.float32)
    b0r = b0.reshape(1, D)
    b1r = b1.reshape(1, D)

    x0 = _mm(Xp, W0)
    E, c0p, c1p = _seg_counts(x0, h01, zf, zc, ones)
    oute = _mid(E, c1p)
    S = _seg_nc(oute, h10, zf)
    X1, res1, x1 = _post0(S, c0p, b0r, Xp, W1)
    E1 = _seg_nc(x1, h01, zf)
    oute1 = _mid(E1, c1p)
    S1 = _seg_nc(oute1, h10, zf)
    res = _post1(S1, c0p, b1r, X1, res1)
    return res[:N_NODES]
